# baseline (speedup 1.0000x reference)
import functools

import jax
import jax.numpy as jnp
from jax import lax
from jax.experimental import pallas as pl
from jax.experimental.pallas import tpu as pltpu
from jax.experimental.pallas import tpu_sc as plsc

N_NODES = 10000
D = 128
N_P = 10240
NC, NS = 2, 16
NW = NC * NS
ROWS_PER_TILE = N_P // NS
CHUNK = 128
CHUNKS = 80
EDGES_P = NW * CHUNKS * CHUNK
ROW_BLK = 1024
GRID = N_P // ROW_BLK

def _sc_mesh():
    return plsc.VectorSubcoreMesh(
        core_axis_name="c", subcore_axis_name="s", num_cores=NC, num_subcores=NS)


def _deg_body(dst_hbm, zeros1_hbm, out_hbm, didx, ones_v, hist):
    cid = lax.axis_index("c")
    sid = lax.axis_index("s")
    wid = sid * NC + cid
    row0 = sid * ROWS_PER_TILE
    pltpu.sync_copy(zeros1_hbm.at[pl.ds(row0, ROWS_PER_TILE)],
                    hist.at[pl.ds(row0, ROWS_PER_TILE)])
    pltpu.sync_copy(dst_hbm.at[wid], didx)
    for i in range(CHUNK // 16):
        ones_v[pl.ds(i * 16, 16)] = jnp.ones((16,), jnp.float32)
    plsc.subcore_barrier()

    def body(j, carry):
        pltpu.sync_copy(ones_v, hist.at[didx.at[j]], add=True)
        return carry

    lax.fori_loop(0, CHUNKS, body, 0)
    plsc.subcore_barrier()
    pltpu.sync_copy(hist.at[pl.ds(row0, ROWS_PER_TILE)],
                    out_hbm.at[cid].at[pl.ds(row0, ROWS_PER_TILE)])


@functools.cache
def _deg_call():
    return pl.kernel(
        _deg_body,
        out_type=jax.ShapeDtypeStruct((NC, N_P), jnp.float32),
        mesh=_sc_mesh(),
        scratch_types=[
            pltpu.VMEM((CHUNKS, CHUNK), jnp.int32),
            pltpu.VMEM((CHUNK,), jnp.float32),
            pltpu.VMEM_SHARED((N_P,), jnp.float32),
        ],
    )


def _agg_body(m_hbm, src_hbm, dst_hbm, zeros2_hbm, out_hbm,
              srcb, dstb, gbuf0, gbuf1, acc,
              isem0, isem1, gsem0, gsem1, ssem0, ssem1):
    cid = lax.axis_index("c")
    sid = lax.axis_index("s")
    wid = sid * NC + cid
    row0 = sid * ROWS_PER_TILE
    pltpu.sync_copy(zeros2_hbm.at[pl.ds(row0, ROWS_PER_TILE)],
                    acc.at[pl.ds(row0, ROWS_PER_TILE)])

    gbufs = (gbuf0, gbuf1)
    isems = (isem0, isem1)
    gsems = (gsem0, gsem1)
    ssems = (ssem0, ssem1)

    def fire_idx(j, slot, sem):
        pltpu.async_copy(src_hbm.at[wid].at[j], srcb.at[slot], sem)
        pltpu.async_copy(dst_hbm.at[wid].at[j], dstb.at[slot], sem)

    def wait_idx(slot, sem):
        pltpu.make_async_copy(src_hbm.at[wid].at[0], srcb.at[slot], sem).wait()
        pltpu.make_async_copy(dst_hbm.at[wid].at[0], dstb.at[slot], sem).wait()

    def wait_gather(s):
        pltpu.make_async_copy(m_hbm.at[srcb.at[0]], gbufs[s], gsems[s]).wait()

    def wait_scatter(s):
        pltpu.make_async_copy(gbufs[s], acc.at[dstb.at[0]], ssems[s]).wait()

    def half(j, s, i4, first):
        sp = 1 - s
        wait_gather(s)
        pltpu.async_copy(gbufs[s], acc.at[dstb.at[i4]], ssems[s], add=True)
        wait_idx((i4 + 1) % 4, isems[sp])
        if not first:
            wait_scatter(sp)
        pltpu.async_copy(m_hbm.at[srcb.at[(i4 + 1) % 4]], gbufs[sp], gsems[sp])
        j2 = jnp.where(j + 2 < CHUNKS, j + 2, j + 2 - CHUNKS)
        fire_idx(j2, (i4 + 2) % 4, isems[s])

    fire_idx(0, 0, isem0)
    fire_idx(1, 1, isem1)
    plsc.subcore_barrier()
    wait_idx(0, isem0)
    pltpu.async_copy(m_hbm.at[srcb.at[0]], gbuf0, gsem0)
    half(0, 0, 0, True)
    half(1, 1, 1, False)

    def body(jj, carry):
        j0 = 2 * jj
        half(j0, 0, j0 % 4, False)
        half(j0 + 1, 1, (j0 + 1) % 4, False)
        return carry

    lax.fori_loop(1, CHUNKS // 2, body, 0)
    wait_idx(1, isem1)
    wait_gather(0)
    wait_scatter(1)
    plsc.subcore_barrier()
    pltpu.sync_copy(acc.at[pl.ds(row0, ROWS_PER_TILE)],
                    out_hbm.at[cid].at[pl.ds(row0, ROWS_PER_TILE)])


@functools.cache
def _agg_call():
    return pl.kernel(
        _agg_body,
        out_type=jax.ShapeDtypeStruct((NC, N_P, D), jnp.float32),
        mesh=_sc_mesh(),
        scratch_types=[
            pltpu.VMEM((4, CHUNK), jnp.int32),
            pltpu.VMEM((4, CHUNK), jnp.int32),
            pltpu.VMEM((CHUNK, D), jnp.float32),
            pltpu.VMEM((CHUNK, D), jnp.float32),
            pltpu.VMEM_SHARED((N_P, D), jnp.float32),
            pltpu.SemaphoreType.DMA,
            pltpu.SemaphoreType.DMA,
            pltpu.SemaphoreType.DMA,
            pltpu.SemaphoreType.DMA,
            pltpu.SemaphoreType.DMA,
            pltpu.SemaphoreType.DMA,
        ],
    )


def _dinv_bcast(deg0, deg1):
    deg = deg0 + deg1 + 1.0
    dinv = lax.rsqrt(deg)
    a = jnp.broadcast_to(dinv[None, :], (D, dinv.shape[0]))
    b = jnp.full((D, D), 1.0 / D, jnp.float32)
    return lax.dot_general(a, b, (((0,), (0,)), ((), ())),
                           preferred_element_type=jnp.float32)


def _tc1a_body(x_ref, w_ref, h_ref):
    h_ref[...] = jnp.dot(x_ref[...], w_ref[...],
                         preferred_element_type=jnp.float32)


def _tc1b_body(deg0_ref, deg1_ref, h_ref, m_ref, dinv_ref):
    dinvb = _dinv_bcast(deg0_ref[0, 0], deg1_ref[0, 0])
    dinv_ref[...] = dinvb
    m_ref[...] = dinvb * h_ref[...]


def _tc2_body(p0_ref, p1_ref, m_ref, dinv_ref, b_ref, w_ref, out_ref):
    s = p0_ref[0] + p1_ref[0] + m_ref[...]
    a = jnp.maximum(dinv_ref[...] * s + b_ref[...], 0.0)
    h = jnp.dot(a, w_ref[...], preferred_element_type=jnp.float32)
    out_ref[...] = dinv_ref[...] * h


def _tc3_body(p0_ref, p1_ref, m_ref, dinv_ref, b_ref, w_ref, bfc_ref, out_ref):
    s = p0_ref[0] + p1_ref[0] + m_ref[...]
    a = jnp.maximum(dinv_ref[...] * s + b_ref[...], 0.0)
    out_ref[...] = jnp.dot(a, w_ref[...],
                           preferred_element_type=jnp.float32) + bfc_ref[...]


_row_spec = pl.BlockSpec((ROW_BLK, D), lambda i: (i, 0))
_p0_spec = pl.BlockSpec((1, ROW_BLK, D), lambda i: (0, i, 0))
_p1_spec = pl.BlockSpec((1, ROW_BLK, D), lambda i: (1, i, 0))
_deg0_spec = pl.BlockSpec((1, 1, ROW_BLK), lambda i: (0, 0, i))
_deg1_spec = pl.BlockSpec((1, 1, ROW_BLK), lambda i: (1, 0, i))
_w_spec = pl.BlockSpec((D, D), lambda i: (0, 0))
_b_spec = pl.BlockSpec((1, D), lambda i: (0, 0))

_tc1a_call = pl.pallas_call(
    _tc1a_body,
    grid=(GRID,),
    in_specs=[_row_spec, _w_spec],
    out_specs=_row_spec,
    out_shape=jax.ShapeDtypeStruct((N_P, D), jnp.float32),
)

_tc1b_call = pl.pallas_call(
    _tc1b_body,
    grid=(GRID,),
    in_specs=[_deg0_spec, _deg1_spec, _row_spec],
    out_specs=[_row_spec, _row_spec],
    out_shape=[jax.ShapeDtypeStruct((N_P, D), jnp.float32),
               jax.ShapeDtypeStruct((N_P, D), jnp.float32)],
)

_tc2_call = pl.pallas_call(
    _tc2_body,
    grid=(GRID,),
    in_specs=[_p0_spec, _p1_spec, _row_spec, _row_spec, _b_spec, _w_spec],
    out_specs=_row_spec,
    out_shape=jax.ShapeDtypeStruct((N_P, D), jnp.float32),
)

_tc3_call = pl.pallas_call(
    _tc3_body,
    grid=(GRID,),
    in_specs=[_p0_spec, _p1_spec, _row_spec, _row_spec, _b_spec, _w_spec,
              _b_spec],
    out_specs=pl.BlockSpec((ROW_BLK, D), lambda i: (i, 0)),
    out_shape=jax.ShapeDtypeStruct((N_NODES, D), jnp.float32),
)


def kernel(x, edge_index, W1, b1, W2, b2, Wfc, bfc):
    e = jnp.asarray(edge_index, jnp.int32)
    n_pad = EDGES_P - e.shape[1]
    k = jnp.arange(n_pad, dtype=jnp.int32)
    src_p = jnp.concatenate([e[0], k % N_NODES]).reshape(NW, CHUNKS, CHUNK)
    dst_p = jnp.concatenate([e[1], N_NODES + k % (N_P - N_NODES)]
                            ).reshape(NW, CHUNKS, CHUNK)

    zeros1 = jnp.zeros((N_P,), jnp.float32)
    zeros2 = jnp.zeros((N_P, D), jnp.float32)
    b1r = b1.reshape(1, D)
    b2r = b2.reshape(1, D)
    bfcr = bfc.reshape(1, D)

    deg = _deg_call()(dst_p, zeros1).reshape(NC, 1, N_P)
    h1 = _tc1a_call(x, W1)
    m1, dinvb = _tc1b_call(deg, deg, h1)
    p1 = _agg_call()(m1, src_p, dst_p, zeros2)
    m2 = _tc2_call(p1, p1, m1, dinvb, b1r, W2)
    p2 = _agg_call()(m2, src_p, dst_p, zeros2)
    return _tc3_call(p2, p2, m2, dinvb, b2r, Wfc, bfcr)
